# baseline (speedup 1.0000x reference)
import functools

import jax
import jax.numpy as jnp
from jax import lax
from jax.experimental import pallas as pl
from jax.experimental.pallas import tpu as pltpu
from jax.experimental.pallas import tpu_sc as plsc

D = 128
V0, V1, V2 = 5, 6, 2
NCOMB = V0 * V1 * V2
TROWS = 64
NC, NS, L = 2, 16, 16
NW = NC * NS
CH = 80
NBUF = 5


def _table_body(w0_ref, w1_ref, w2_ref, t_ref):
    for c in range(NCOMB):
        i, r = divmod(c, V1 * V2)
        j, k = divmod(r, V2)
        t_ref[pl.ds(c, 1), :] = (
            w0_ref[pl.ds(i, 1), :] + w1_ref[pl.ds(j, 1), :] + w2_ref[pl.ds(k, 1), :]
        )
    t_ref[pl.ds(NCOMB, TROWS - NCOMB), :] = jnp.zeros((TROWS - NCOMB, D), jnp.float32)


def _build_table(W0, W1, W2):
    return pl.pallas_call(
        _table_body,
        out_shape=jax.ShapeDtypeStruct((TROWS, D), jnp.float32),
    )(W0, W1, W2)


def _sc_body(epw, t_hbm, a0_hbm, a1_hbm, a2_hbm, out_hbm, *scr):
    tvm = scr[0]
    ia0 = scr[1:1 + NBUF]
    ia1 = scr[1 + NBUF:1 + 2 * NBUF]
    ia2 = scr[1 + 2 * NBUF:1 + 3 * NBUF]
    rows = scr[1 + 3 * NBUF:1 + 4 * NBUF]
    tsem = scr[1 + 4 * NBUF]
    isem = scr[2 + 4 * NBUF:2 + 5 * NBUF]
    osem = scr[2 + 5 * NBUF:2 + 6 * NBUF]

    nch = epw // CH
    wid = lax.axis_index("s") * NC + lax.axis_index("c")
    base = wid * epw

    tcp = pltpu.async_copy(t_hbm, tvm, tsem)
    for b in range(NBUF):
        off = base + b * CH
        pltpu.async_copy(a0_hbm.at[pl.ds(off, CH)], ia0[b], isem[b])
        pltpu.async_copy(a1_hbm.at[pl.ds(off, CH)], ia1[b], isem[b])
        pltpu.async_copy(a2_hbm.at[pl.ds(off, CH)], ia2[b], isem[b])
    tcp.wait()

    def outer(g, carry):
        k0 = g * NBUF
        for b in range(NBUF):
            k = k0 + b
            for col in (a0_hbm, a1_hbm, a2_hbm):
                pltpu.make_async_copy(
                    col.at[pl.ds(0, CH)], ia0[b], isem[b]
                ).wait()
            @pl.when(k >= NBUF)
            def _():
                pltpu.make_async_copy(
                    rows[b], out_hbm.at[pl.ds(0, CH * D)], osem[b]
                ).wait()

            @plsc.parallel_loop(0, CH // L, 1, unroll=5)
            def group(gg):
                s = gg * L
                cv = (
                    ia0[b][pl.ds(s, L)] * (V1 * V2 * D)
                    + ia1[b][pl.ds(s, L)] * (V2 * D)
                    + ia2[b][pl.ds(s, L)] * D
                )
                prev = None
                prev_eo = 0
                for l in range(L):
                    off = cv[l]
                    eo = (s + l) * D
                    cur = []
                    for i, v in enumerate(range(0, D, L)):
                        cur.append(tvm[pl.ds(off + v, L)])
                        if prev is not None:
                            rows[b][pl.ds(prev_eo + v, L)] = prev[i]
                    prev, prev_eo = cur, eo
                for i, v in enumerate(range(0, D, L)):
                    rows[b][pl.ds(prev_eo + v, L)] = prev[i]

            pltpu.async_copy(
                rows[b], out_hbm.at[pl.ds((base + k * CH) * D, CH * D)], osem[b]
            )

            @pl.when(k + NBUF < nch)
            def _():
                off = base + (k + NBUF) * CH
                pltpu.async_copy(a0_hbm.at[pl.ds(off, CH)], ia0[b], isem[b])
                pltpu.async_copy(a1_hbm.at[pl.ds(off, CH)], ia1[b], isem[b])
                pltpu.async_copy(a2_hbm.at[pl.ds(off, CH)], ia2[b], isem[b])
        return carry

    lax.fori_loop(0, nch // NBUF, outer, 0)

    for b in range(NBUF):
        pltpu.make_async_copy(
            rows[b], out_hbm.at[pl.ds(0, CH * D)], osem[b]
        ).wait()


def _sc_lookup(t_flat, a0, a1, a2):
    n = a0.shape[0]
    assert n % (NW * CH) == 0 and (n // NW) % (CH * NBUF) == 0
    epw = n // NW
    mesh = plsc.VectorSubcoreMesh(core_axis_name="c", subcore_axis_name="s")
    scratch = (
        [pltpu.VMEM((TROWS * D,), jnp.float32)]
        + [pltpu.VMEM((CH,), jnp.int32) for _ in range(3 * NBUF)]
        + [pltpu.VMEM((CH * D,), jnp.float32) for _ in range(NBUF)]
        + [pltpu.SemaphoreType.DMA]
        + [pltpu.SemaphoreType.DMA for _ in range(2 * NBUF)]
    )
    return pl.kernel(
        functools.partial(_sc_body, epw),
        out_type=jax.ShapeDtypeStruct((n * D,), jnp.float32),
        mesh=mesh,
        scratch_types=scratch,
        compiler_params=pltpu.CompilerParams(needs_layout_passes=False),
    )(t_flat, a0, a1, a2)


def kernel(edge_attr, W0, W1, W2):
    n = edge_attr.shape[0]
    a = edge_attr.astype(jnp.int32)
    a0, a1, a2 = a[:, 0], a[:, 1], a[:, 2]
    t = _build_table(W0, W1, W2).reshape(TROWS * D)
    return _sc_lookup(t, a0, a1, a2).reshape(n, D)


# speedup vs baseline: 1.3558x; 1.3558x over previous
import functools

import jax
import jax.numpy as jnp
from jax import lax
from jax.experimental import pallas as pl
from jax.experimental.pallas import tpu as pltpu
from jax.experimental.pallas import tpu_sc as plsc

D = 128
V0, V1, V2 = 5, 6, 2
NCOMB = V0 * V1 * V2
TROWS = 64
NC, NS, L = 2, 16, 16
NW = NC * NS
CH = 80
NBUF = 5


def _table_body(w0_ref, w1_ref, w2_ref, t_ref):
    for c in range(NCOMB):
        i, r = divmod(c, V1 * V2)
        j, k = divmod(r, V2)
        t_ref[pl.ds(c, 1), :] = (
            w0_ref[pl.ds(i, 1), :] + w1_ref[pl.ds(j, 1), :] + w2_ref[pl.ds(k, 1), :]
        )
    t_ref[pl.ds(NCOMB, TROWS - NCOMB), :] = jnp.zeros((TROWS - NCOMB, D), jnp.float32)


def _build_table(W0, W1, W2):
    return pl.pallas_call(
        _table_body,
        out_shape=jax.ShapeDtypeStruct((TROWS, D), jnp.float32),
    )(W0, W1, W2)


def _sc_body(epw, t_hbm, a0_hbm, a1_hbm, a2_hbm, out_hbm, *scr):
    tvm = scr[0]
    ia0 = scr[1:1 + NBUF]
    ia1 = scr[1 + NBUF:1 + 2 * NBUF]
    ia2 = scr[1 + 2 * NBUF:1 + 3 * NBUF]
    rows = scr[1 + 3 * NBUF:1 + 4 * NBUF]
    tsem = scr[1 + 4 * NBUF]
    isem = scr[2 + 4 * NBUF:2 + 5 * NBUF]
    osem = scr[2 + 5 * NBUF:2 + 6 * NBUF]

    nch = epw // CH
    wid = lax.axis_index("s") * NC + lax.axis_index("c")
    base = wid * epw

    tcp = pltpu.async_copy(t_hbm, tvm, tsem)
    for b in range(NBUF):
        off = base + b * CH
        pltpu.async_copy(a0_hbm.at[pl.ds(off, CH)], ia0[b], isem[b])
        pltpu.async_copy(a1_hbm.at[pl.ds(off, CH)], ia1[b], isem[b])
        pltpu.async_copy(a2_hbm.at[pl.ds(off, CH)], ia2[b], isem[b])
    tcp.wait()

    def outer(g, carry):
        k0 = g * NBUF
        for b in range(NBUF):
            k = k0 + b
            for col in (a0_hbm, a1_hbm, a2_hbm):
                pltpu.make_async_copy(
                    col.at[pl.ds(0, CH)], ia0[b], isem[b]
                ).wait()
            @pl.when(k >= NBUF)
            def _():
                pltpu.make_async_copy(
                    rows[b], out_hbm.at[pl.ds(0, CH * D)], osem[b]
                ).wait()

            @plsc.parallel_loop(0, CH // L, 1, unroll=1)
            def group(gg):
                s = gg * L
                cv = (
                    ia0[b][pl.ds(s, L)] * (V1 * V2 * D)
                    + ia1[b][pl.ds(s, L)] * (V2 * D)
                    + ia2[b][pl.ds(s, L)] * D
                )
                prev = None
                prev_eo = 0
                for l in range(L):
                    off = cv[l]
                    eo = (s + l) * D
                    cur = []
                    for i, v in enumerate(range(0, D, L)):
                        cur.append(tvm[pl.ds(off + v, L)])
                        if prev is not None:
                            rows[b][pl.ds(prev_eo + v, L)] = prev[i]
                    prev, prev_eo = cur, eo
                for i, v in enumerate(range(0, D, L)):
                    rows[b][pl.ds(prev_eo + v, L)] = prev[i]

            pltpu.async_copy(
                rows[b], out_hbm.at[pl.ds((base + k * CH) * D, CH * D)], osem[b]
            )

            @pl.when(k + NBUF < nch)
            def _():
                off = base + (k + NBUF) * CH
                pltpu.async_copy(a0_hbm.at[pl.ds(off, CH)], ia0[b], isem[b])
                pltpu.async_copy(a1_hbm.at[pl.ds(off, CH)], ia1[b], isem[b])
                pltpu.async_copy(a2_hbm.at[pl.ds(off, CH)], ia2[b], isem[b])
        return carry

    lax.fori_loop(0, nch // NBUF, outer, 0)

    for b in range(NBUF):
        pltpu.make_async_copy(
            rows[b], out_hbm.at[pl.ds(0, CH * D)], osem[b]
        ).wait()


def _sc_lookup(t_flat, a0, a1, a2):
    n = a0.shape[0]
    assert n % (NW * CH) == 0 and (n // NW) % (CH * NBUF) == 0
    epw = n // NW
    mesh = plsc.VectorSubcoreMesh(core_axis_name="c", subcore_axis_name="s")
    scratch = (
        [pltpu.VMEM((TROWS * D,), jnp.float32)]
        + [pltpu.VMEM((CH,), jnp.int32) for _ in range(3 * NBUF)]
        + [pltpu.VMEM((CH * D,), jnp.float32) for _ in range(NBUF)]
        + [pltpu.SemaphoreType.DMA]
        + [pltpu.SemaphoreType.DMA for _ in range(2 * NBUF)]
    )
    return pl.kernel(
        functools.partial(_sc_body, epw),
        out_type=jax.ShapeDtypeStruct((n * D,), jnp.float32),
        mesh=mesh,
        scratch_types=scratch,
        compiler_params=pltpu.CompilerParams(needs_layout_passes=False),
    )(t_flat, a0, a1, a2)


def kernel(edge_attr, W0, W1, W2):
    n = edge_attr.shape[0]
    a = edge_attr.astype(jnp.int32)
    a0, a1, a2 = a[:, 0], a[:, 1], a[:, 2]
    t = _build_table(W0, W1, W2).reshape(TROWS * D)
    return _sc_lookup(t, a0, a1, a2).reshape(n, D)
